# baseline (speedup 1.0000x reference)
import functools

import jax
import jax.numpy as jnp
import numpy as np
from jax import lax
from jax.experimental import pallas as pl
from jax.experimental.pallas import tpu as pltpu
from jax.experimental.pallas import tpu_sc as plsc

VOCAB_ = 100000
EMB_ = 64
HID_ = 64
B_ = 1024
L_ = 50

_NC = 2
_NS = 16
_NW = _NC * _NS

_N_IDX = B_ * L_
_CHUNK = 100
_CHUNKS_TOTAL = _N_IDX // _CHUNK
_CHUNKS_PER_W = _CHUNKS_TOTAL // _NW


def _sc_gather_body(idx_hbm, table_hbm, out_hbm, idx_v, rows_v, sem):
    wid = lax.axis_index("s") * _NC + lax.axis_index("c")
    base = wid * _CHUNKS_PER_W
    pltpu.sync_copy(idx_hbm.at[pl.ds(base, _CHUNKS_PER_W)], idx_v)
    for j in range(_CHUNKS_PER_W):
        pltpu.async_copy(table_hbm.at[idx_v.at[j]], rows_v.at[j], sem)
    pltpu.make_async_copy(out_hbm.at[pl.ds(base, _CHUNKS_PER_W)], rows_v, sem).wait()
    pltpu.sync_copy(rows_v, out_hbm.at[pl.ds(base, _CHUNKS_PER_W)])


@functools.lru_cache(maxsize=None)
def _sc_gather():
    return pl.kernel(
        _sc_gather_body,
        out_type=jax.ShapeDtypeStruct((_CHUNKS_TOTAL, _CHUNK, EMB_), jnp.float32),
        mesh=plsc.VectorSubcoreMesh(core_axis_name="c", subcore_axis_name="s"),
        scratch_types=[
            pltpu.VMEM((_CHUNKS_PER_W, _CHUNK), jnp.int32),
            pltpu.VMEM((_CHUNKS_PER_W, _CHUNK, EMB_), jnp.float32),
            pltpu.SemaphoreType.DMA,
        ],
        compiler_params=pltpu.CompilerParams(use_tc_tiling_on_sc=False),
    )


_LP = L_ // 2
_W2 = 2 * HID_


def _rnn_body(x_ref, wih_ref, whh_ref, b_ref, out_ref):
    x_all = x_ref[...].reshape(B_ * _LP, _W2)
    a = jnp.dot(x_all, wih_ref[...], preferred_element_type=jnp.float32)
    out_ref[...] = (a + b_ref[...]).reshape(B_, _LP, _W2)

    whh = whh_ref[...]

    def step(c, h):
        ac = out_ref[:, c, :]
        h1 = jnp.tanh(ac[:, :HID_]
                      + jnp.dot(h, whh, preferred_element_type=jnp.float32))
        h2 = jnp.tanh(ac[:, HID_:]
                      + jnp.dot(h1, whh, preferred_element_type=jnp.float32))
        out_ref[:, c, :] = jnp.concatenate([h1, h2], axis=1)
        return h2

    lax.fori_loop(0, _LP, step, jnp.zeros((B_, HID_), jnp.float32))


_rnn = pl.pallas_call(
    _rnn_body,
    out_shape=jax.ShapeDtypeStruct((B_, _LP, _W2), jnp.float32),
)


def _blockdiag2(w):
    z = jnp.zeros((HID_, HID_), w.dtype)
    return jnp.block([[w, z], [z, w]])


def kernel(sentences, emb_table, W_ih, W_hh, b_ih, b_hh):
    idx = sentences.astype(jnp.int32).reshape(_CHUNKS_TOTAL, _CHUNK)
    x = _sc_gather()(idx, emb_table)
    x2 = x.reshape(B_, _LP, _W2)
    bias1 = b_ih + b_hh
    bias2 = jnp.concatenate([bias1, bias1]).reshape(1, _W2)
    ys2 = _rnn(x2, _blockdiag2(W_ih.T), W_hh.T, bias2)
    final_output = ys2.reshape(B_, L_, HID_)
    h = ys2[:, _LP - 1, HID_:][None, :, :]
    return final_output, h


# speedup vs baseline: 1.5550x; 1.5550x over previous
import functools

import jax
import jax.numpy as jnp
from jax import lax
from jax.experimental import pallas as pl
from jax.experimental.pallas import tpu as pltpu
from jax.experimental.pallas import tpu_sc as plsc

VOCAB_ = 100000
EMB_ = 64
HID_ = 64
B_ = 1024
L_ = 50

_NC = 2
_NS = 16
_NW = _NC * _NS
_BW = B_ // _NW


def _sc_gather_body(idx_hbm, table_hbm, out_hbm, blk_v, idx_v, rows_v, sem):
    wid = lax.axis_index("s") * _NC + lax.axis_index("c")
    b0 = wid * _BW
    pltpu.sync_copy(idx_hbm.at[pl.ds(b0, _BW)], blk_v)

    def tr(g, carry):
        q = lax.iota(jnp.int32, 16) + g * 16
        vals = plsc.load_gather(blk_v, [q & (_BW - 1), q >> 5])
        idx_v[pl.ds(pl.multiple_of(g * 16, 16), 16)] = vals
        return carry

    lax.fori_loop(0, _BW * L_ // 16, tr, 0)

    def gth(j, carry):
        pltpu.async_copy(
            table_hbm.at[idx_v.at[pl.ds(pl.multiple_of(j * _BW, _BW), _BW)]],
            rows_v.at[j], sem)
        return carry

    lax.fori_loop(0, L_, gth, 0)
    pltpu.make_async_copy(out_hbm.at[:, pl.ds(b0, _BW)], rows_v, sem).wait()
    pltpu.sync_copy(rows_v, out_hbm.at[:, pl.ds(b0, _BW)])


@functools.lru_cache(maxsize=None)
def _sc_gather():
    return pl.kernel(
        _sc_gather_body,
        out_type=jax.ShapeDtypeStruct((L_, B_, EMB_), jnp.float32),
        mesh=plsc.VectorSubcoreMesh(core_axis_name="c", subcore_axis_name="s"),
        scratch_types=[
            pltpu.VMEM((_BW, L_), jnp.int32),
            pltpu.VMEM((_BW * L_,), jnp.int32),
            pltpu.VMEM((L_, _BW, EMB_), jnp.float32),
            pltpu.SemaphoreType.DMA,
        ],
        compiler_params=pltpu.CompilerParams(
            use_tc_tiling_on_sc=False, needs_layout_passes=False),
    )


_BP = B_ // 2
_W2 = 2 * HID_


def _rnn_body(x_ref, wih_ref, whh_ref, b_ref, out_ref):
    x_all = x_ref[...].reshape(L_ * _BP, _W2)
    a = jnp.dot(x_all, wih_ref[...], preferred_element_type=jnp.float32)
    out_ref[...] = (a + b_ref[...]).reshape(L_, _BP, _W2)

    def step(t, h):
        hn = jnp.tanh(
            out_ref[t]
            + jnp.dot(h, whh_ref[...], preferred_element_type=jnp.float32)
        )
        out_ref[t] = hn
        return hn

    lax.fori_loop(0, L_, step, jnp.zeros((_BP, _W2), jnp.float32))


_rnn = pl.pallas_call(
    _rnn_body,
    out_shape=jax.ShapeDtypeStruct((L_, _BP, _W2), jnp.float32),
)


def _blockdiag2(w):
    z = jnp.zeros((HID_, HID_), w.dtype)
    return jnp.block([[w, z], [z, w]])


def kernel(sentences, emb_table, W_ih, W_hh, b_ih, b_hh):
    idx = sentences.astype(jnp.int32)
    x = _sc_gather()(idx, emb_table)
    x2 = x.reshape(L_, _BP, _W2)
    bias1 = b_ih + b_hh
    bias2 = jnp.concatenate([bias1, bias1]).reshape(1, _W2)
    ys2 = _rnn(x2, _blockdiag2(W_ih.T), _blockdiag2(W_hh.T), bias2)
    ys = ys2.reshape(L_, B_, HID_)
    final_output = jnp.swapaxes(ys, 0, 1)
    h = ys[L_ - 1][None, :, :]
    return final_output, h
